# Initial kernel scaffold; baseline (speedup 1.0000x reference)
import jax
import jax.numpy as jnp
from jax.experimental import pallas as pl


def kernel(token_ids, W):
    raise NotImplementedError("write your pallas kernel here")


# speedup vs baseline: 4.7164x; 4.7164x over previous
import functools

import jax
import jax.numpy as jnp
from jax import lax
from jax.experimental import pallas as pl
from jax.experimental.pallas import tpu as pltpu
from jax.experimental.pallas import tpu_sc as plsc

NUM_CORES = 2
NUM_SUBCORES = 16
NUM_WORKERS = NUM_CORES * NUM_SUBCORES
CHUNK = 128


@functools.partial(jax.jit, static_argnums=(2, 3))
def _sc_gather(table, idx_grouped, groups_per_worker, feat):
    total_rows = idx_grouped.shape[0] * CHUNK
    mesh = plsc.VectorSubcoreMesh(core_axis_name="c", subcore_axis_name="s")

    @functools.partial(
        pl.kernel,
        mesh=mesh,
        compiler_params=pltpu.CompilerParams(use_tc_tiling_on_sc=False),
        out_type=jax.ShapeDtypeStruct((total_rows, feat), jnp.int32),
        scratch_types=[
            pltpu.VMEM((groups_per_worker, CHUNK), jnp.int32),
            pltpu.VMEM((2, CHUNK, feat), jnp.int32),
            pltpu.SemaphoreType.DMA,
            pltpu.SemaphoreType.DMA,
        ],
    )
    def run(table_hbm, idx_hbm, out_hbm, idx_v, rows_v, gsem, osem):
        wid = lax.axis_index("s") * NUM_CORES + lax.axis_index("c")
        gbase = wid * groups_per_worker
        rbase = gbase * CHUNK
        pltpu.sync_copy(idx_hbm.at[pl.ds(gbase, groups_per_worker)], idx_v)

        pltpu.async_copy(table_hbm.at[idx_v.at[0]], rows_v.at[0], gsem)

        def body(g, _):
            slot = lax.rem(g, 2)
            nxt = lax.rem(g + 1, 2)

            @pl.when(g > 0)
            def _():
                pltpu.make_async_copy(
                    rows_v.at[nxt],
                    out_hbm.at[pl.ds(rbase + (g - 1) * CHUNK, CHUNK)],
                    osem,
                ).wait()

            @pl.when(g + 1 < groups_per_worker)
            def _():
                pltpu.async_copy(
                    table_hbm.at[idx_v.at[g + 1]], rows_v.at[nxt], gsem
                )

            pltpu.make_async_copy(
                table_hbm.at[idx_v.at[g]], rows_v.at[slot], gsem
            ).wait()

            pltpu.async_copy(
                rows_v.at[slot],
                out_hbm.at[pl.ds(rbase + g * CHUNK, CHUNK)],
                osem,
            )
            return 0

        lax.fori_loop(0, groups_per_worker, body, 0, unroll=2)
        last = lax.rem(groups_per_worker - 1, 2)
        pltpu.make_async_copy(
            rows_v.at[last],
            out_hbm.at[pl.ds(rbase + (groups_per_worker - 1) * CHUNK, CHUNK)],
            osem,
        ).wait()

    return run(table, idx_grouped)


def kernel(token_ids, W):
    B, T = token_ids.shape
    V, feat = W.shape
    n = B * T
    assert n % (NUM_WORKERS * CHUNK) == 0
    groups_per_worker = n // (NUM_WORKERS * CHUNK)
    table_i32 = W.astype(jnp.int32)
    idx = token_ids.astype(jnp.int32).reshape(n // CHUNK, CHUNK)
    rows = _sc_gather(table_i32, idx, groups_per_worker, feat)
    return rows.reshape(B, T, feat).astype(jnp.int64)


# speedup vs baseline: 5.1880x; 1.1000x over previous
import functools

import jax
import jax.numpy as jnp
from jax import lax
from jax.experimental import pallas as pl
from jax.experimental.pallas import tpu as pltpu
from jax.experimental.pallas import tpu_sc as plsc

NUM_CORES = 2
NUM_SUBCORES = 16
NUM_WORKERS = NUM_CORES * NUM_SUBCORES
CHUNK = 512


@functools.partial(jax.jit, static_argnums=(2, 3))
def _sc_gather(table, idx_grouped, groups_per_worker, feat):
    total_rows = idx_grouped.shape[0] * CHUNK
    mesh = plsc.VectorSubcoreMesh(core_axis_name="c", subcore_axis_name="s")

    @functools.partial(
        pl.kernel,
        mesh=mesh,
        compiler_params=pltpu.CompilerParams(use_tc_tiling_on_sc=False),
        out_type=jax.ShapeDtypeStruct((total_rows, feat), jnp.int32),
        scratch_types=[
            pltpu.VMEM((groups_per_worker, CHUNK), jnp.int32),
            pltpu.VMEM((2, CHUNK, feat), jnp.int32),
            pltpu.SemaphoreType.DMA,
            pltpu.SemaphoreType.DMA,
        ],
    )
    def run(table_hbm, idx_hbm, out_hbm, idx_v, rows_v, gsem, osem):
        wid = lax.axis_index("s") * NUM_CORES + lax.axis_index("c")
        gbase = wid * groups_per_worker
        rbase = gbase * CHUNK
        pltpu.sync_copy(idx_hbm.at[pl.ds(gbase, groups_per_worker)], idx_v)

        pltpu.async_copy(table_hbm.at[idx_v.at[0]], rows_v.at[0], gsem)

        def body(g, _):
            slot = lax.rem(g, 2)
            nxt = lax.rem(g + 1, 2)

            @pl.when(g > 0)
            def _():
                pltpu.make_async_copy(
                    rows_v.at[nxt],
                    out_hbm.at[pl.ds(rbase + (g - 1) * CHUNK, CHUNK)],
                    osem,
                ).wait()

            @pl.when(g + 1 < groups_per_worker)
            def _():
                pltpu.async_copy(
                    table_hbm.at[idx_v.at[g + 1]], rows_v.at[nxt], gsem
                )

            pltpu.make_async_copy(
                table_hbm.at[idx_v.at[g]], rows_v.at[slot], gsem
            ).wait()

            pltpu.async_copy(
                rows_v.at[slot],
                out_hbm.at[pl.ds(rbase + g * CHUNK, CHUNK)],
                osem,
            )
            return 0

        lax.fori_loop(0, groups_per_worker, body, 0, unroll=2)
        last = lax.rem(groups_per_worker - 1, 2)
        pltpu.make_async_copy(
            rows_v.at[last],
            out_hbm.at[pl.ds(rbase + (groups_per_worker - 1) * CHUNK, CHUNK)],
            osem,
        ).wait()

    return run(table, idx_grouped)


def kernel(token_ids, W):
    B, T = token_ids.shape
    V, feat = W.shape
    n = B * T
    assert n % (NUM_WORKERS * CHUNK) == 0
    groups_per_worker = n // (NUM_WORKERS * CHUNK)
    table_i32 = W.astype(jnp.int32)
    idx = token_ids.astype(jnp.int32).reshape(n // CHUNK, CHUNK)
    rows = _sc_gather(table_i32, idx, groups_per_worker, feat)
    return rows.reshape(B, T, feat).astype(jnp.int64)


# speedup vs baseline: 5.2755x; 1.0169x over previous
import functools

import jax
import jax.numpy as jnp
from jax import lax
from jax.experimental import pallas as pl
from jax.experimental.pallas import tpu as pltpu
from jax.experimental.pallas import tpu_sc as plsc

NUM_CORES = 2
NUM_SUBCORES = 16
NUM_WORKERS = NUM_CORES * NUM_SUBCORES
DEPTH = 4


@functools.partial(jax.jit, static_argnums=(2, 3, 4))
def _sc_gather(table, token_ids, B, T, feat):
    b_per_w = B // NUM_WORKERS
    mesh = plsc.VectorSubcoreMesh(core_axis_name="c", subcore_axis_name="s")

    @functools.partial(
        pl.kernel,
        mesh=mesh,
        compiler_params=pltpu.CompilerParams(use_tc_tiling_on_sc=False),
        out_type=jax.ShapeDtypeStruct((B, T, feat), jnp.int32),
        scratch_types=[
            pltpu.VMEM((b_per_w, T), jnp.int32),
            pltpu.VMEM((DEPTH, T, feat), jnp.int32),
        ]
        + [pltpu.SemaphoreType.DMA] * (2 * DEPTH),
    )
    def run(table_hbm, idx_hbm, out_hbm, idx_v, rows_v, *sems):
        gsems = sems[:DEPTH]
        wsems = sems[DEPTH:]
        wid = lax.axis_index("s") * NUM_CORES + lax.axis_index("c")
        base = wid * b_per_w
        pltpu.sync_copy(idx_hbm.at[pl.ds(base, b_per_w)], idx_v)

        for d in range(DEPTH - 1):
            pltpu.async_copy(table_hbm.at[idx_v.at[d]], rows_v.at[d], gsems[d])

        def outer(o, _):
            for d in range(DEPTH):
                g = o * DEPTH + d
                prev = (d - 1) % DEPTH

                @pl.when(g > 0)
                def _():
                    pltpu.make_async_copy(
                        rows_v.at[prev], out_hbm.at[base + g - 1], wsems[prev]
                    ).wait()

                @pl.when(g + DEPTH - 1 < b_per_w)
                def _():
                    pltpu.async_copy(
                        table_hbm.at[idx_v.at[g + DEPTH - 1]],
                        rows_v.at[prev],
                        gsems[prev],
                    )

                pltpu.make_async_copy(
                    table_hbm.at[idx_v.at[g]], rows_v.at[d], gsems[d]
                ).wait()
                pltpu.async_copy(rows_v.at[d], out_hbm.at[base + g], wsems[d])
            return 0

        lax.fori_loop(0, b_per_w // DEPTH, outer, 0)
        last = (b_per_w - 1) % DEPTH
        pltpu.make_async_copy(
            rows_v.at[last], out_hbm.at[base + b_per_w - 1], wsems[last]
        ).wait()

    return run(table, token_ids)


def kernel(token_ids, W):
    B, T = token_ids.shape
    V, feat = W.shape
    assert B % (NUM_WORKERS * DEPTH) == 0
    table_i32 = W.astype(jnp.int32)
    idx = token_ids.astype(jnp.int32)
    out = _sc_gather(table_i32, idx, B, T, feat)
    return out.astype(jnp.int64)


# speedup vs baseline: 8.7364x; 1.6560x over previous
import functools

import jax
import jax.numpy as jnp
from jax import lax
from jax.experimental import pallas as pl
from jax.experimental.pallas import tpu as pltpu
from jax.experimental.pallas import tpu_sc as plsc

NUM_CORES = 2
NUM_SUBCORES = 16
NUM_WORKERS = NUM_CORES * NUM_SUBCORES
DEPTH = 4
PAD = 128


@functools.partial(jax.jit, static_argnums=(2, 3, 4))
def _sc_gather_t(table, idx_t, B, T, feat):
    bpw = B // NUM_WORKERS
    mesh = plsc.VectorSubcoreMesh(core_axis_name="c", subcore_axis_name="s")

    @functools.partial(
        pl.kernel,
        mesh=mesh,
        compiler_params=pltpu.CompilerParams(use_tc_tiling_on_sc=False),
        out_type=jax.ShapeDtypeStruct((B, T, PAD), jnp.int32),
        scratch_types=[
            pltpu.VMEM((T, bpw), jnp.int32),
            pltpu.VMEM((DEPTH, bpw, feat), jnp.int32),
        ]
        + [pltpu.SemaphoreType.DMA] * (2 * DEPTH),
    )
    def run(table_hbm, idx_hbm, out_hbm, idx_v, rows_v, *sems):
        gsems = sems[:DEPTH]
        wsems = sems[DEPTH:]
        wid = lax.axis_index("s") * NUM_CORES + lax.axis_index("c")
        b0 = wid * bpw
        pltpu.sync_copy(idx_hbm.at[:, pl.ds(b0, bpw)], idx_v)

        for d in range(DEPTH - 1):
            pltpu.async_copy(table_hbm.at[idx_v.at[d]], rows_v.at[d], gsems[d])

        def outer(o, _):
            for d in range(DEPTH):
                t = o * DEPTH + d
                prev = (d - 1) % DEPTH

                @pl.when(t > 0)
                def _():
                    pltpu.make_async_copy(
                        rows_v.at[prev],
                        out_hbm.at[pl.ds(b0, bpw), t - 1, pl.ds(0, feat)],
                        wsems[prev],
                    ).wait()

                @pl.when(t + DEPTH - 1 < T)
                def _():
                    pltpu.async_copy(
                        table_hbm.at[idx_v.at[t + DEPTH - 1]],
                        rows_v.at[prev],
                        gsems[prev],
                    )

                pltpu.make_async_copy(
                    table_hbm.at[idx_v.at[t]], rows_v.at[d], gsems[d]
                ).wait()
                pltpu.async_copy(
                    rows_v.at[d],
                    out_hbm.at[pl.ds(b0, bpw), t, pl.ds(0, feat)],
                    wsems[d],
                )
            return 0

        lax.fori_loop(0, T // DEPTH, outer, 0)
        last = (T - 1) % DEPTH
        pltpu.make_async_copy(
            rows_v.at[last],
            out_hbm.at[pl.ds(b0, bpw), T - 1, pl.ds(0, feat)],
            wsems[last],
        ).wait()

    return run(table, idx_t)


def kernel(token_ids, W):
    B, T = token_ids.shape
    V, feat = W.shape
    assert B % (NUM_WORKERS * 128) == 0 and T % DEPTH == 0
    table_i32 = W.astype(jnp.int32)
    idx_t = token_ids.astype(jnp.int32).T
    padded = _sc_gather_t(table_i32, idx_t, B, T, feat)
    return padded[:, :, :feat].astype(jnp.int64)


# speedup vs baseline: 8.9917x; 1.0292x over previous
import functools

import jax
import jax.numpy as jnp
from jax import lax
from jax.experimental import pallas as pl
from jax.experimental.pallas import tpu as pltpu
from jax.experimental.pallas import tpu_sc as plsc

NUM_CORES = 2
NUM_SUBCORES = 16
NUM_WORKERS = NUM_CORES * NUM_SUBCORES
DEPTH = 5
PAD = 128


@functools.partial(jax.jit, static_argnums=(2, 3, 4))
def _sc_gather_t(table, idx_t, B, T, feat):
    bpw = B // NUM_WORKERS
    mesh = plsc.VectorSubcoreMesh(core_axis_name="c", subcore_axis_name="s")

    @functools.partial(
        pl.kernel,
        mesh=mesh,
        compiler_params=pltpu.CompilerParams(use_tc_tiling_on_sc=False),
        out_type=jax.ShapeDtypeStruct((B, T, PAD), jnp.int32),
        scratch_types=[
            pltpu.VMEM((T, bpw), jnp.int32),
            pltpu.VMEM((DEPTH, bpw, feat), jnp.int32),
        ]
        + [pltpu.SemaphoreType.DMA] * (2 * DEPTH),
    )
    def run(table_hbm, idx_hbm, out_hbm, idx_v, rows_v, *sems):
        gsems = sems[:DEPTH]
        wsems = sems[DEPTH:]
        wid = lax.axis_index("s") * NUM_CORES + lax.axis_index("c")
        b0 = wid * bpw
        pltpu.sync_copy(idx_hbm.at[:, pl.ds(b0, bpw)], idx_v)

        for d in range(DEPTH - 1):
            pltpu.async_copy(table_hbm.at[idx_v.at[d]], rows_v.at[d], gsems[d])

        def outer(o, _):
            for d in range(DEPTH):
                t = o * DEPTH + d
                prev = (d - 1) % DEPTH

                @pl.when(t > 0)
                def _():
                    pltpu.make_async_copy(
                        rows_v.at[prev],
                        out_hbm.at[pl.ds(b0, bpw), t - 1, pl.ds(0, feat)],
                        wsems[prev],
                    ).wait()

                @pl.when(t + DEPTH - 1 < T)
                def _():
                    pltpu.async_copy(
                        table_hbm.at[idx_v.at[t + DEPTH - 1]],
                        rows_v.at[prev],
                        gsems[prev],
                    )

                pltpu.make_async_copy(
                    table_hbm.at[idx_v.at[t]], rows_v.at[d], gsems[d]
                ).wait()
                pltpu.async_copy(
                    rows_v.at[d],
                    out_hbm.at[pl.ds(b0, bpw), t, pl.ds(0, feat)],
                    wsems[d],
                )
            return 0

        lax.fori_loop(0, T // DEPTH, outer, 0)
        last = (T - 1) % DEPTH
        pltpu.make_async_copy(
            rows_v.at[last],
            out_hbm.at[pl.ds(b0, bpw), T - 1, pl.ds(0, feat)],
            wsems[last],
        ).wait()

    return run(table, idx_t)


def kernel(token_ids, W):
    B, T = token_ids.shape
    V, feat = W.shape
    assert B % (NUM_WORKERS * 128) == 0 and T % DEPTH == 0
    table_i32 = W.astype(jnp.int32)
    idx_t = token_ids.astype(jnp.int32).T
    padded = _sc_gather_t(table_i32, idx_t, B, T, feat)
    return padded[:, :, :feat].astype(jnp.int64)


# speedup vs baseline: 9.2739x; 1.0314x over previous
import functools

import jax
import jax.numpy as jnp
from jax import lax
from jax.experimental import pallas as pl
from jax.experimental.pallas import tpu as pltpu
from jax.experimental.pallas import tpu_sc as plsc

NUM_CORES = 2
NUM_SUBCORES = 16
NUM_WORKERS = NUM_CORES * NUM_SUBCORES
DEPTH = 8
PAD = 128


@functools.partial(jax.jit, static_argnums=(2, 3, 4))
def _sc_gather_t(table, idx_t, B, T, feat):
    bpw = B // NUM_WORKERS
    mesh = plsc.VectorSubcoreMesh(core_axis_name="c", subcore_axis_name="s")

    @functools.partial(
        pl.kernel,
        mesh=mesh,
        compiler_params=pltpu.CompilerParams(use_tc_tiling_on_sc=False),
        out_type=jax.ShapeDtypeStruct((B, T, PAD), jnp.int32),
        scratch_types=[
            pltpu.VMEM((T, bpw), jnp.int32),
            pltpu.VMEM((DEPTH, bpw, feat), jnp.int32),
        ]
        + [pltpu.SemaphoreType.DMA] * (2 * DEPTH),
    )
    def run(table_hbm, idx_hbm, out_hbm, idx_v, rows_v, *sems):
        gsems = sems[:DEPTH]
        wsems = sems[DEPTH:]
        wid = lax.axis_index("s") * NUM_CORES + lax.axis_index("c")
        b0 = wid * bpw
        pltpu.sync_copy(idx_hbm.at[:, pl.ds(b0, bpw)], idx_v)

        for d in range(DEPTH - 1):
            pltpu.async_copy(table_hbm.at[idx_v.at[d]], rows_v.at[d], gsems[d])

        def outer(o, _):
            for d in range(DEPTH):
                t = o * DEPTH + d
                prev = (d - 1) % DEPTH

                @pl.when(t > 0)
                def _():
                    pltpu.make_async_copy(
                        rows_v.at[prev],
                        out_hbm.at[pl.ds(b0, bpw), t - 1, pl.ds(0, feat)],
                        wsems[prev],
                    ).wait()

                @pl.when(t + DEPTH - 1 < T)
                def _():
                    pltpu.async_copy(
                        table_hbm.at[idx_v.at[t + DEPTH - 1]],
                        rows_v.at[prev],
                        gsems[prev],
                    )

                pltpu.make_async_copy(
                    table_hbm.at[idx_v.at[t]], rows_v.at[d], gsems[d]
                ).wait()
                pltpu.async_copy(
                    rows_v.at[d],
                    out_hbm.at[pl.ds(b0, bpw), t, pl.ds(0, feat)],
                    wsems[d],
                )
            return 0

        lax.fori_loop(0, T // DEPTH, outer, 0)
        last = (T - 1) % DEPTH
        pltpu.make_async_copy(
            rows_v.at[last],
            out_hbm.at[pl.ds(b0, bpw), T - 1, pl.ds(0, feat)],
            wsems[last],
        ).wait()

    return run(table, idx_t)


def kernel(token_ids, W):
    B, T = token_ids.shape
    V, feat = W.shape
    assert B % (NUM_WORKERS * 128) == 0 and T % DEPTH == 0
    table_i32 = W.astype(jnp.int32)
    idx_t = token_ids.astype(jnp.int32).T
    padded = _sc_gather_t(table_i32, idx_t, B, T, feat)
    return padded[:, :, :feat].astype(jnp.int64)


# speedup vs baseline: 9.2810x; 1.0008x over previous
import functools

import jax
import jax.numpy as jnp
from jax import lax
from jax.experimental import pallas as pl
from jax.experimental.pallas import tpu as pltpu
from jax.experimental.pallas import tpu_sc as plsc

NUM_CORES = 2
NUM_SUBCORES = 16
NUM_WORKERS = NUM_CORES * NUM_SUBCORES
DEPTH = 10
PAD = 128


@functools.partial(jax.jit, static_argnums=(2, 3, 4))
def _sc_gather_t(table, idx_t, B, T, feat):
    bpw = B // NUM_WORKERS
    mesh = plsc.VectorSubcoreMesh(core_axis_name="c", subcore_axis_name="s")

    @functools.partial(
        pl.kernel,
        mesh=mesh,
        compiler_params=pltpu.CompilerParams(use_tc_tiling_on_sc=False),
        out_type=jax.ShapeDtypeStruct((B, T, PAD), jnp.int32),
        scratch_types=[
            pltpu.VMEM((T, bpw), jnp.int32),
            pltpu.VMEM((DEPTH, bpw, feat), jnp.int32),
        ]
        + [pltpu.SemaphoreType.DMA] * (2 * DEPTH),
    )
    def run(table_hbm, idx_hbm, out_hbm, idx_v, rows_v, *sems):
        gsems = sems[:DEPTH]
        wsems = sems[DEPTH:]
        wid = lax.axis_index("s") * NUM_CORES + lax.axis_index("c")
        b0 = wid * bpw
        pltpu.sync_copy(idx_hbm.at[:, pl.ds(b0, bpw)], idx_v)

        for d in range(DEPTH - 1):
            pltpu.async_copy(table_hbm.at[idx_v.at[d]], rows_v.at[d], gsems[d])

        def outer(o, _):
            for d in range(DEPTH):
                t = o * DEPTH + d
                prev = (d - 1) % DEPTH

                @pl.when(t > 0)
                def _():
                    pltpu.make_async_copy(
                        rows_v.at[prev],
                        out_hbm.at[pl.ds(b0, bpw), t - 1, pl.ds(0, feat)],
                        wsems[prev],
                    ).wait()

                @pl.when(t + DEPTH - 1 < T)
                def _():
                    pltpu.async_copy(
                        table_hbm.at[idx_v.at[t + DEPTH - 1]],
                        rows_v.at[prev],
                        gsems[prev],
                    )

                pltpu.make_async_copy(
                    table_hbm.at[idx_v.at[t]], rows_v.at[d], gsems[d]
                ).wait()
                pltpu.async_copy(
                    rows_v.at[d],
                    out_hbm.at[pl.ds(b0, bpw), t, pl.ds(0, feat)],
                    wsems[d],
                )
            return 0

        lax.fori_loop(0, T // DEPTH, outer, 0)
        last = (T - 1) % DEPTH
        pltpu.make_async_copy(
            rows_v.at[last],
            out_hbm.at[pl.ds(b0, bpw), T - 1, pl.ds(0, feat)],
            wsems[last],
        ).wait()

    return run(table, idx_t)


def kernel(token_ids, W):
    B, T = token_ids.shape
    V, feat = W.shape
    assert B % (NUM_WORKERS * 128) == 0 and T % DEPTH == 0
    table_i32 = W.astype(jnp.int32)
    idx_t = token_ids.astype(jnp.int32).T
    padded = _sc_gather_t(table_i32, idx_t, B, T, feat)
    return padded[:, :, :feat].astype(jnp.int64)
